# baseline (speedup 1.0000x reference)
import functools

import jax
import jax.numpy as jnp
import numpy as np
from jax.experimental import pallas as pl
from jax.experimental.pallas import tpu as pltpu


def _kimia_body(x_ref, wikp_ref, wiv_ref, wq_ref, wk1_ref, wk2_ref, wv_ref,
                w1_ref, w2_ref, temb_ref, sins_ref, o_ref, k_scr, v_scr):
    f32 = jnp.float32
    NI = temb_ref.shape[0] - 1
    DK = wq_ref.shape[1]
    scale = np.float32(1.0 / np.sqrt(DK))

    biases = jnp.dot(sins_ref[...], wk2_ref[...], preferred_element_type=f32)

    xb = x_ref[...]
    k_scr[0] = jnp.dot(xb, wikp_ref[...], preferred_element_type=f32)
    v0 = jnp.dot(xb, wiv_ref[...], preferred_element_type=f32)
    v_scr[0] = v0
    A = v0

    for t in range(NI - 1):
        h = jnp.dot(A, w1_ref[...], preferred_element_type=f32) + temb_ref[t]
        Bt = jnp.dot(jax.nn.gelu(h), w2_ref[...], preferred_element_type=f32)
        k_scr[t + 1] = (jnp.dot(Bt, wk1_ref[...], preferred_element_type=f32)
                        + biases[t])
        v_scr[t + 1] = jnp.dot(Bt, wv_ref[...], preferred_element_type=f32)
        q = jnp.dot(Bt, wq_ref[...], preferred_element_type=f32) * scale

        n = t + 2
        svals = [jnp.sum(q * k_scr[j], axis=-1, keepdims=True)
                 for j in range(n)]
        m = svals[0]
        for j in range(1, n):
            m = jnp.maximum(m, svals[j])
        evals = [jnp.exp(s - m) for s in svals]
        den = evals[0]
        for j in range(1, n):
            den = den + evals[j]
        r = 1.0 / den
        A = (evals[0] * r) * v_scr[0]
        for j in range(1, n):
            A = A + (evals[j] * r) * v_scr[j]

    h = jnp.dot(A, w1_ref[...], preferred_element_type=f32) + temb_ref[NI]
    o_ref[...] = jnp.dot(jax.nn.gelu(h), w2_ref[...], preferred_element_type=f32)


@functools.partial(jax.jit, static_argnames=("interpret",))
def kernel(x, Wik, Wiv, Wq, Wk, Wv, W1, W2, t_emb, interpret=False):
    B, D = x.shape
    DK = Wq.shape[1]
    TR = Wk.shape[0] - D
    NI = t_emb.shape[0] - 1
    dt = x.dtype

    wikp = jnp.pad(Wik, ((0, 0), (0, DK - Wik.shape[1])))
    wk1 = Wk[:D]
    wk2 = Wk[D:]
    t_enc = jnp.pi * (0.5 ** jnp.arange(TR, dtype=dt))
    tvals = jnp.arange(NI, dtype=dt)
    sins = jnp.sin(tvals[:, None] * t_enc[None, :])

    BB = 512
    grid = (B // BB,)
    full = lambda shape: pl.BlockSpec(shape, lambda i: tuple(0 for _ in shape))

    return pl.pallas_call(
        _kimia_body,
        out_shape=jax.ShapeDtypeStruct((B, D), dt),
        grid=grid,
        in_specs=[
            pl.BlockSpec((BB, D), lambda i: (i, 0)),
            full((D, DK)),
            full((D, D)),
            full((D, DK)),
            full((D, DK)),
            full((TR, DK)),
            full((D, D)),
            full((D, D)),
            full((D, D)),
            full((NI + 1, D)),
            full((NI, TR)),
        ],
        out_specs=pl.BlockSpec((BB, D), lambda i: (i, 0)),
        scratch_shapes=[
            pltpu.VMEM((NI, BB, DK), jnp.float32),
            pltpu.VMEM((NI, BB, D), jnp.float32),
        ],
        compiler_params=pltpu.CompilerParams(
            dimension_semantics=("parallel",),
            vmem_limit_bytes=56 * 1024 * 1024,
        ),
        name="ind_kimia_fused",
        interpret=interpret,
    )(x, wikp, Wiv, Wq, wk1, wk2, Wv, W1, W2, t_emb, sins)


# speedup vs baseline: 1.1044x; 1.1044x over previous
import functools

import jax
import jax.numpy as jnp
import numpy as np
from jax.experimental import pallas as pl
from jax.experimental.pallas import tpu as pltpu


def _kimia_body(x_ref, xw_ref, wkvq_ref, wk2_ref,
                w1_ref, w2_ref, temb_ref, sins_ref, o_ref, k_scr, v_scr):
    f32 = jnp.float32
    NI = temb_ref.shape[0] - 1
    D = w1_ref.shape[0]
    DK = D

    biases = jnp.dot(sins_ref[...], wk2_ref[...], preferred_element_type=f32)

    kv0 = jnp.dot(x_ref[...], xw_ref[...], preferred_element_type=f32)
    k_scr[0] = kv0[:, :DK]
    v0 = kv0[:, DK:]
    v_scr[0] = v0
    A = v0

    for t in range(NI - 1):
        h = jnp.dot(A, w1_ref[...], preferred_element_type=f32) + temb_ref[t]
        Bt = jnp.dot(jax.nn.gelu(h), w2_ref[...], preferred_element_type=f32)
        kvq = jnp.dot(Bt, wkvq_ref[...], preferred_element_type=f32)
        k_scr[t + 1] = kvq[:, :DK] + biases[t]
        v_scr[t + 1] = kvq[:, DK:DK + D]
        q = kvq[:, DK + D:]

        n = t + 2
        svals = [jnp.sum(q * k_scr[j], axis=-1, keepdims=True)
                 for j in range(n)]
        m = svals[0]
        for j in range(1, n):
            m = jnp.maximum(m, svals[j])
        evals = [jnp.exp(s - m) for s in svals]
        den = evals[0]
        for j in range(1, n):
            den = den + evals[j]
        r = 1.0 / den
        A = (evals[0] * r) * v_scr[0]
        for j in range(1, n):
            A = A + (evals[j] * r) * v_scr[j]

    h = jnp.dot(A, w1_ref[...], preferred_element_type=f32) + temb_ref[NI]
    o_ref[...] = jnp.dot(jax.nn.gelu(h), w2_ref[...], preferred_element_type=f32)


@functools.partial(jax.jit, static_argnames=("interpret",))
def kernel(x, Wik, Wiv, Wq, Wk, Wv, W1, W2, t_emb, interpret=False):
    B, D = x.shape
    DK = Wq.shape[1]
    TR = Wk.shape[0] - D
    NI = t_emb.shape[0] - 1
    dt = x.dtype

    scale = np.float32(1.0 / np.sqrt(DK))
    wikp = jnp.pad(Wik, ((0, 0), (0, DK - Wik.shape[1])))
    xw = jnp.concatenate([wikp, Wiv], axis=1)
    wkvq = jnp.concatenate([Wk[:D], Wv, Wq * scale], axis=1)
    wk2 = Wk[D:]
    t_enc = jnp.pi * (0.5 ** jnp.arange(TR, dtype=dt))
    tvals = jnp.arange(NI, dtype=dt)
    sins = jnp.sin(tvals[:, None] * t_enc[None, :])

    BB = 256
    grid = (B // BB,)
    full = lambda shape: pl.BlockSpec(shape, lambda i: tuple(0 for _ in shape))

    return pl.pallas_call(
        _kimia_body,
        out_shape=jax.ShapeDtypeStruct((B, D), dt),
        grid=grid,
        in_specs=[
            pl.BlockSpec((BB, D), lambda i: (i, 0)),
            full((D, DK + D)),
            full((D, DK + D + DK)),
            full((TR, DK)),
            full((D, D)),
            full((D, D)),
            full((NI + 1, D)),
            full((NI, TR)),
        ],
        out_specs=pl.BlockSpec((BB, D), lambda i: (i, 0)),
        scratch_shapes=[
            pltpu.VMEM((NI, BB, DK), jnp.float32),
            pltpu.VMEM((NI, BB, D), jnp.float32),
        ],
        compiler_params=pltpu.CompilerParams(
            dimension_semantics=("parallel",),
            vmem_limit_bytes=56 * 1024 * 1024,
        ),
        name="ind_kimia_fused",
        interpret=interpret,
    )(x, xw, wkvq, wk2, W1, W2, t_emb, sins)
